# baseline (speedup 1.0000x reference)
import jax
import jax.numpy as jnp
from jax.experimental import pallas as pl
from jax.experimental.pallas import tpu as pltpu

_CBLK = 2048


def _body(uf_ref, uid_ref, hist_ref, wuf_ref, buf_ref, wut_ref, but_ref,
          corpus_ref, scores_ref, ue_ref):
    @pl.when(pl.program_id(0) == 0)
    def _():
        ufe = jax.lax.dot_general(
            uf_ref[...], wuf_ref[...], (((1,), (1,)), ((), ())),
            preferred_element_type=jnp.float32) + buf_ref[...][None, :]
        ti = jnp.concatenate([uid_ref[...], ufe, hist_ref[...]], axis=1)
        ue_ref[...] = jax.lax.dot_general(
            ti, wut_ref[...], (((1,), (1,)), ((), ())),
            preferred_element_type=jnp.float32) + but_ref[...][None, :]

    scores_ref[...] = jax.lax.dot_general(
        ue_ref[...], corpus_ref[...], (((1,), (1,)), ((), ())),
        preferred_element_type=jnp.float32)


def kernel(user_id, user_features, user_history, user_id_table, item_id_table,
           W_uf, b_uf, W_ut, b_ut, item_corpus_emb, num_items):
    B = user_id.shape[0]
    C, DI = item_corpus_emb.shape
    DU = W_uf.shape[0]

    hist_sum = jnp.mean(jnp.take(item_id_table, user_history, axis=0), axis=1)
    uid_emb = jnp.take(user_id_table, user_id, axis=0)

    nblk = pl.cdiv(C, _CBLK)
    full = lambda shape: pl.BlockSpec(shape, lambda j: (0,) * len(shape))
    scores = pl.pallas_call(
        _body,
        grid=(nblk,),
        in_specs=[
            full((B, user_features.shape[1])),
            full((B, DU)),
            full((B, DI)),
            full(W_uf.shape),
            full(b_uf.shape),
            full(W_ut.shape),
            full(b_ut.shape),
            pl.BlockSpec((_CBLK, DI), lambda j: (j, 0)),
        ],
        out_specs=pl.BlockSpec((B, _CBLK), lambda j: (0, j)),
        out_shape=jax.ShapeDtypeStruct((B, C), jnp.float32),
        scratch_shapes=[pltpu.VMEM((B, DU), jnp.float32)],
    )(user_features, uid_emb, hist_sum, W_uf, b_uf, W_ut, b_ut,
      item_corpus_emb)

    top_values, top_indices = jax.lax.top_k(scores, 100)
    return top_values, top_indices
